# Initial kernel scaffold; baseline (speedup 1.0000x reference)
import jax
import jax.numpy as jnp
from jax.experimental import pallas as pl


def kernel(node_embeddings, edge_index, batch, W1, b1, W2, b2, gamma1, beta1, W3, b3, W4, b4, gamma2, beta2, Wf1, bf1, Wf2, bf2):
    raise NotImplementedError("write your pallas kernel here")


# speedup vs baseline: 10.4321x; 10.4321x over previous
import functools

import jax
import jax.numpy as jnp
from jax import lax
from jax.experimental import pallas as pl
from jax.experimental.pallas import tpu as pltpu
from jax.experimental.pallas import tpu_sc as plsc

_N = 10000
_DIM = 32
_GRAPHS = 64
_BN_EPS = 1e-3

_NCORES = 2
_NSUB = 16
_NW = _NCORES * _NSUB
_CHUNK = 128
_NACC = 10112
_RPT = _NACC // _NSUB
_DUMMY = _N


def _matmul(a, b):
    return lax.dot_general(a, b, (((1,), (0,)), ((), ())),
                           preferred_element_type=jnp.float32)


def _proj_body(x_ref, w_ref, o_ref):
    o_ref[...] = _matmul(x_ref[...], w_ref[...])


def _mid_body(y_ref, p_ref, b1_ref, w2_ref, b2_ref, g_ref, be_ref, w3_ref,
              o_ref):
    h = y_ref[...] + p_ref[0, :_N, :] + p_ref[1, :_N, :] + b1_ref[...]
    h = jnp.maximum(h, 0.0)
    z = jnp.maximum(_matmul(h, w2_ref[...]) + b2_ref[...], 0.0)
    m = jnp.mean(z, axis=0, keepdims=True)
    v = jnp.mean((z - m) ** 2, axis=0, keepdims=True)
    z = (z - m) * lax.rsqrt(v + _BN_EPS) * g_ref[...] + be_ref[...]
    x1 = jnp.maximum(z, 0.0)
    o_ref[...] = _matmul(x1, w3_ref[...])


def _head_body(y_ref, p_ref, b3_ref, w4_ref, b4_ref, g_ref, be_ref,
               batch_ref, wf1_ref, bf1_ref, wf2_ref, bf2_ref, o_ref):
    h = y_ref[...] + p_ref[0, :_N, :] + p_ref[1, :_N, :] + b3_ref[...]
    h = jnp.maximum(h, 0.0)
    z = jnp.maximum(_matmul(h, w4_ref[...]) + b4_ref[...], 0.0)
    m = jnp.mean(z, axis=0, keepdims=True)
    v = jnp.mean((z - m) ** 2, axis=0, keepdims=True)
    z = (z - m) * lax.rsqrt(v + _BN_EPS) * g_ref[...] + be_ref[...]
    x2 = jnp.maximum(z, 0.0)
    gids = lax.broadcasted_iota(jnp.int32, (_GRAPHS, _N), 0)
    onehot = (gids == batch_ref[...]).astype(jnp.float32)
    pooled = _matmul(onehot, x2)
    t = jnp.maximum(_matmul(pooled, wf1_ref[...]) + bf1_ref[...], 0.0)
    logits = _matmul(t, wf2_ref[...]) + bf2_ref[...]
    mx = jnp.max(logits, axis=-1, keepdims=True)
    e = jnp.exp(logits - mx)
    o_ref[...] = e / jnp.sum(e, axis=-1, keepdims=True)


def _make_seg_sum(nchunk):
    mesh = plsc.VectorSubcoreMesh(core_axis_name="c", subcore_axis_name="s")

    @functools.partial(
        pl.kernel,
        out_type=jax.ShapeDtypeStruct((_NCORES, _NACC, _DIM), jnp.float32),
        mesh=mesh,
        scratch_types=[
            pltpu.VMEM((nchunk, _CHUNK), jnp.int32),
            pltpu.VMEM((nchunk, _CHUNK), jnp.int32),
            pltpu.VMEM((_CHUNK, _DIM), jnp.float32),
            pltpu.VMEM_SHARED((_NACC, _DIM), jnp.float32),
        ],
        compiler_params=pltpu.CompilerParams(use_tc_tiling_on_sc=False),
    )
    def seg(y_hbm, src_hbm, dst_hbm, zeros_hbm, out_hbm,
            src_v, dst_v, rows_v, acc):
        c = lax.axis_index("c")
        s = lax.axis_index("s")
        wid = c * _NSUB + s
        r0 = pl.multiple_of(s * _RPT, 8)
        pltpu.sync_copy(zeros_hbm.at[pl.ds(r0, _RPT)],
                        acc.at[pl.ds(r0, _RPT)])
        pltpu.sync_copy(src_hbm.at[wid], src_v)
        pltpu.sync_copy(dst_hbm.at[wid], dst_v)
        plsc.subcore_barrier()

        def body(j, carry):
            pltpu.sync_copy(y_hbm.at[src_v.at[j]], rows_v)
            pltpu.sync_copy(rows_v, acc.at[dst_v.at[j]], add=True)
            return carry

        lax.fori_loop(0, nchunk, body, 0)
        plsc.subcore_barrier()
        pltpu.sync_copy(acc.at[pl.ds(r0, _RPT)],
                        out_hbm.at[c, pl.ds(r0, _RPT)])

    return seg


def kernel(node_embeddings, edge_index, batch, W1, b1, W2, b2, gamma1,
           beta1, W3, b3, W4, b4, gamma2, beta2, Wf1, bf1, Wf2, bf2):
    e = edge_index.shape[1]
    nchunk = -(-e // (_NW * _CHUNK))
    epad = _NW * nchunk * _CHUNK

    src = jnp.concatenate(
        [edge_index[0], jnp.zeros((epad - e,), jnp.int32)]
    ).reshape(_NW, nchunk, _CHUNK)
    dst = jnp.concatenate(
        [edge_index[1], jnp.full((epad - e,), _DUMMY, jnp.int32)]
    ).reshape(_NW, nchunk, _CHUNK)
    zeros = jnp.zeros((_NACC, _DIM), jnp.float32)

    seg = _make_seg_sum(nchunk)

    proj = pl.pallas_call(
        _proj_body,
        out_shape=jax.ShapeDtypeStruct((_N, _DIM), jnp.float32),
    )
    mid = pl.pallas_call(
        _mid_body,
        out_shape=jax.ShapeDtypeStruct((_N, _DIM), jnp.float32),
    )
    head = pl.pallas_call(
        _head_body,
        out_shape=jax.ShapeDtypeStruct((_GRAPHS, 2), jnp.float32),
    )

    y1 = proj(node_embeddings, W1)
    p1 = seg(y1, src, dst, zeros)
    y2 = mid(y1, p1, b1.reshape(1, -1), W2, b2.reshape(1, -1),
             gamma1.reshape(1, -1), beta1.reshape(1, -1), W3)
    p2 = seg(y2, src, dst, zeros)
    out = head(y2, p2, b3.reshape(1, -1), W4, b4.reshape(1, -1),
               gamma2.reshape(1, -1), beta2.reshape(1, -1),
               batch.reshape(1, -1), Wf1, bf1.reshape(1, -1),
               Wf2, bf2.reshape(1, -1))
    return out
